# baseline (device time: 30198 ns/iter reference)
import jax
import jax.numpy as jnp
from jax import lax
from jax.experimental import pallas as pl
from jax.experimental.pallas import tpu as pltpu

N_DEV = 16
M = 1024
K = 512
N = 512

STREAMS = ((0, 768, (1, 4)), (768, 256, (4, 1)))
N_EXCH = 24
RS_ROWS = 3 * (160 + 40 + 96 + 24)


def kernel(t, W):
    def body(t_ref, w_ref, out_ref, stage_ref, comm_ref, ag_ref,
             send_sems, recv_sems):
        i = lax.axis_index("i")

        def group(u):
            g = lax.rem(lax.div(i, u), 4)
            return g, i - g * u

        barrier_sem = pltpu.get_barrier_semaphore()
        for u in (1, 4):
            g, gbase = group(u)
            for d in (1, 2, 3):
                peer = gbase + lax.rem(g + d, 4) * u
                pl.semaphore_signal(
                    barrier_sem, inc=1,
                    device_id=(peer,),
                    device_id_type=pl.DeviceIdType.MESH,
                )
        pl.semaphore_wait(barrier_sem, 6)

        def mm(row_lo, rows):
            out_ref[pl.ds(row_lo, rows), :] = jnp.dot(
                t_ref[pl.ds(row_lo, rows), :], w_ref[...],
                preferred_element_type=jnp.float32,
            )

        lo = [jnp.int32(base) for base, _, _ in STREAMS]
        pending = [None, None]
        ctr = {"sem": 0, "comm": 0, "stage": 0}

        def start(s, slot):
            _, R, units = STREAMS[s]
            sem_base = ctr["sem"]
            ctr["sem"] += 3
            if slot < 2:
                H = R // 4 if slot == 0 else R // 16
                u = units[slot]
                g, gbase = group(u)
                keep_lo = lo[s] + g * H
                comm_base = ctr["comm"]
                ctr["comm"] += 3 * H
                rdmas = []
                for d in (1, 2, 3):
                    jm = lax.rem(g + d, 4)
                    peer = gbase + jm * u
                    src_off = ctr["stage"]
                    ctr["stage"] += H
                    send_lo = lo[s] + jm * H
                    stage_ref[pl.ds(src_off, H), :] = out_ref[
                        pl.ds(send_lo, H), :
                    ].astype(jnp.bfloat16)
                    r = 4 - d
                    rdma = pltpu.make_async_remote_copy(
                        src_ref=stage_ref.at[pl.ds(src_off, H), :],
                        dst_ref=comm_ref.at[
                            pl.ds(comm_base + (r - 1) * H, H), :
                        ],
                        send_sem=send_sems.at[sem_base + r - 1],
                        recv_sem=recv_sems.at[sem_base + r - 1],
                        device_id=(peer,),
                        device_id_type=pl.DeviceIdType.MESH,
                    )
                    rdma.start()
                    rdmas.append(rdma)
                lo[s] = keep_lo
                pending[s] = (rdmas, keep_lo, comm_base, slot, H)
            else:
                H = R // 16 if slot == 2 else R // 4
                u = units[3 - slot]
                g, gbase = group(u)
                rdmas = []
                for d in (1, 2, 3):
                    peer = gbase + lax.rem(g + d, 4) * u
                    r = 4 - d
                    rdma = pltpu.make_async_remote_copy(
                        src_ref=ag_ref.at[pl.ds(lo[s], H), :],
                        dst_ref=ag_ref.at[pl.ds(lo[s], H), :],
                        send_sem=send_sems.at[sem_base + r - 1],
                        recv_sem=recv_sems.at[sem_base + r - 1],
                        device_id=(peer,),
                        device_id_type=pl.DeviceIdType.MESH,
                    )
                    rdma.start()
                    rdmas.append(rdma)
                pending[s] = (rdmas, lo[s] - g * H, g, slot, H)
                lo[s] = lo[s] - g * H

        def finish(s):
            rdmas, aux, aux2, slot, H = pending[s]
            for rdma in rdmas:
                rdma.wait()
            if slot < 2:
                keep_lo, comm_base = aux, aux2
                out_ref[pl.ds(keep_lo, H), :] += (
                    comm_ref[pl.ds(comm_base, H), :].astype(jnp.float32)
                    + comm_ref[pl.ds(comm_base + H, H), :].astype(
                        jnp.float32
                    )
                    + comm_ref[pl.ds(comm_base + 2 * H, H), :].astype(
                        jnp.float32
                    )
                )
                if slot == 1:
                    ag_ref[pl.ds(keep_lo, H), :] = out_ref[
                        pl.ds(keep_lo, H), :
                    ].astype(jnp.bfloat16)
            else:
                block_lo, g = aux, aux2
                if slot == 2:
                    out_ref[pl.ds(block_lo, 4 * H), :] = ag_ref[
                        pl.ds(block_lo, 4 * H), :
                    ].astype(jnp.float32)
                else:
                    for d in (1, 2, 3):
                        jm = lax.rem(g + d, 4)
                        out_ref[pl.ds(block_lo + jm * H, H), :] = ag_ref[
                            pl.ds(block_lo + jm * H, H), :
                        ].astype(jnp.float32)

        def start0(s, d, sem_base, comm_base, stage_base, ginfo):
            base, R, units = STREAMS[s]
            H = R // 4
            g, gbase = ginfo
            jm = lax.rem(g + d, 4)
            peer = gbase + jm * u0(s)
            send_lo = base + jm * H
            mm(send_lo, H)
            stage_ref[pl.ds(stage_base + (d - 1) * H, H), :] = out_ref[
                pl.ds(send_lo, H), :
            ].astype(jnp.bfloat16)
            r = 4 - d
            rdma = pltpu.make_async_remote_copy(
                src_ref=stage_ref.at[pl.ds(stage_base + (d - 1) * H, H), :],
                dst_ref=comm_ref.at[pl.ds(comm_base + (r - 1) * H, H), :],
                send_sem=send_sems.at[sem_base + r - 1],
                recv_sem=recv_sems.at[sem_base + r - 1],
                device_id=(peer,),
                device_id_type=pl.DeviceIdType.MESH,
            )
            rdma.start()
            return rdma

        def u0(s):
            return STREAMS[s][2][0]

        slot0 = []
        for s in range(2):
            base, R, units = STREAMS[s]
            H = R // 4
            g, gbase = group(u0(s))
            sem_base = ctr["sem"]
            ctr["sem"] += 3
            comm_base = ctr["comm"]
            ctr["comm"] += 3 * H
            stage_base = ctr["stage"]
            ctr["stage"] += 3 * H
            slot0.append((sem_base, comm_base, stage_base, (g, gbase), []))
        for d in (1, 2, 3):
            for s in range(2):
                sem_base, comm_base, stage_base, ginfo, rdmas = slot0[s]
                rdmas.append(
                    start0(s, d, sem_base, comm_base, stage_base, ginfo)
                )
        for s in range(2):
            base, R, units = STREAMS[s]
            H = R // 4
            sem_base, comm_base, stage_base, (g, gbase), rdmas = slot0[s]
            keep_lo = base + g * H
            mm(keep_lo, H)
            lo[s] = keep_lo
            pending[s] = (rdmas, keep_lo, comm_base, 0, H)

        for slot in range(1, 4):
            for s in range(2):
                finish(s)
                start(s, slot)
        for s in range(2):
            finish(s)

    return pl.pallas_call(
        body,
        out_shape=jax.ShapeDtypeStruct((M, N), jnp.float32),
        in_specs=[
            pl.BlockSpec(memory_space=pltpu.VMEM),
            pl.BlockSpec(memory_space=pltpu.VMEM),
        ],
        out_specs=pl.BlockSpec(memory_space=pltpu.VMEM),
        scratch_shapes=[
            pltpu.VMEM((RS_ROWS, N), jnp.bfloat16),
            pltpu.VMEM((RS_ROWS, N), jnp.bfloat16),
            pltpu.VMEM((M, N), jnp.bfloat16),
            pltpu.SemaphoreType.DMA((N_EXCH,)),
            pltpu.SemaphoreType.DMA((N_EXCH,)),
        ],
        compiler_params=pltpu.CompilerParams(collective_id=0),
    )(t, W)


# device time: 27888 ns/iter; 1.0828x vs baseline; 1.0828x over previous
import jax
import jax.numpy as jnp
from jax import lax
from jax.experimental import pallas as pl
from jax.experimental.pallas import tpu as pltpu

N_DEV = 16
M = 1024
K = 512
N = 512

STREAMS = ((0, 640, (1, 4)), (640, 384, (4, 1)))
N_EXCH = 42
RS_ROWS = 3 * (160 + 40 + 96 + 24)


def kernel(t, W):
    def body(t_ref, w_ref, out_ref, stage_ref, comm_ref, ag_ref,
             send_sems, recv_sems):
        i = lax.axis_index("i")

        def group(u):
            g = lax.rem(lax.div(i, u), 4)
            return g, i - g * u

        def peer_at(u, g, gbase, d):
            return gbase + lax.rem(g + d, 4) * u

        barrier_sem = pltpu.get_barrier_semaphore()
        for u in (1, 4):
            g, gbase = group(u)
            for d in (1, 2, 3):
                pl.semaphore_signal(
                    barrier_sem, inc=1,
                    device_id=(peer_at(u, g, gbase, d),),
                    device_id_type=pl.DeviceIdType.MESH,
                )
        pl.semaphore_wait(barrier_sem, 6)

        def mm(row_lo, rows):
            out_ref[pl.ds(row_lo, rows), :] = jnp.dot(
                t_ref[pl.ds(row_lo, rows), :], w_ref[...],
                preferred_element_type=jnp.float32,
            )

        ctr = {"sem": 0, "comm": 0, "stage": 0}
        pending = [None, None]

        def rs_send(s, block_lo, H, u, g, gbase, sem_base, comm_base,
                    d, fuse_mm):
            jm = lax.rem(g + d, 4)
            send_lo = block_lo + jm * H
            if fuse_mm:
                mm(send_lo, H)
            src_off = ctr["stage"]
            ctr["stage"] += H
            stage_ref[pl.ds(src_off, H), :] = out_ref[
                pl.ds(send_lo, H), :
            ].astype(jnp.bfloat16)
            r = 4 - d
            rdma = pltpu.make_async_remote_copy(
                src_ref=stage_ref.at[pl.ds(src_off, H), :],
                dst_ref=comm_ref.at[pl.ds(comm_base + (r - 1) * H, H), :],
                send_sem=send_sems.at[sem_base + r - 1],
                recv_sem=recv_sems.at[sem_base + r - 1],
                device_id=(peer_at(u, g, gbase, d),),
                device_id_type=pl.DeviceIdType.MESH,
            )
            rdma.start()
            return rdma

        def rs_finish(s):
            rdmas, keep_lo, comm_base, H = pending[s]
            for rdma in rdmas:
                rdma.wait()
            out_ref[pl.ds(keep_lo, H), :] += (
                comm_ref[pl.ds(comm_base, H), :].astype(jnp.float32)
                + comm_ref[pl.ds(comm_base + H, H), :].astype(jnp.float32)
                + comm_ref[pl.ds(comm_base + 2 * H, H), :].astype(
                    jnp.float32
                )
            )
            return keep_lo, H

        def ag_send(src_lo, H, u, g, gbase, sem_base, d):
            r = 4 - d
            rdma = pltpu.make_async_remote_copy(
                src_ref=ag_ref.at[pl.ds(src_lo, H), :],
                dst_ref=ag_ref.at[pl.ds(src_lo, H), :],
                send_sem=send_sems.at[sem_base + r - 1],
                recv_sem=recv_sems.at[sem_base + r - 1],
                device_id=(peer_at(u, g, gbase, d),),
                device_id_type=pl.DeviceIdType.MESH,
            )
            rdma.start()
            return rdma

        st0 = []
        for s, (base, R, units) in enumerate(STREAMS):
            g, gbase = group(units[0])
            sem_base = ctr["sem"]
            ctr["sem"] += 3
            comm_base = ctr["comm"]
            ctr["comm"] += 3 * (R // 4)
            st0.append([g, gbase, sem_base, comm_base, []])
        for d in (1, 2, 3):
            for s, (base, R, units) in enumerate(STREAMS):
                g, gbase, sem_base, comm_base, rdmas = st0[s]
                rdmas.append(
                    rs_send(s, base, R // 4, units[0], g,
                            gbase, sem_base, comm_base, d, True)
                )
        for s, (base, R, units) in enumerate(STREAMS):
            g, gbase, sem_base, comm_base, rdmas = st0[s]
            keep_lo = base + g * (R // 4)
            mm(keep_lo, R // 4)
            pending[s] = (rdmas, keep_lo, comm_base, R // 4)

        for s, (base, R, units) in enumerate(STREAMS):
            block_lo, H0 = rs_finish(s)
            H1 = R // 16
            g, gbase = group(units[1])
            sem_base = ctr["sem"]
            ctr["sem"] += 3
            comm_base = ctr["comm"]
            ctr["comm"] += 3 * H1
            rdmas = [
                rs_send(s, block_lo, H1, units[1], g, gbase, sem_base,
                        comm_base, d, False)
                for d in (1, 2, 3)
            ]
            pending[s] = (rdmas, block_lo + g * H1, comm_base, H1)

        ag_ctx = [None, None]
        waiters = [[], []]
        for s, (base, R, units) in enumerate(STREAMS):
            chunk_lo, H1 = rs_finish(s)
            ag_ref[pl.ds(chunk_lo, H1), :] = out_ref[
                pl.ds(chunk_lo, H1), :
            ].astype(jnp.bfloat16)
            g1, gbase1 = group(units[1])
            block2_lo = chunk_lo - g1 * H1
            sem_base = ctr["sem"]
            ctr["sem"] += 3
            a2a = [
                ag_send(chunk_lo, H1, units[1], g1, gbase1, sem_base, d)
                for d in (1, 2, 3)
            ]
            g0, gbase0 = group(units[0])
            sem_base = ctr["sem"]
            ctr["sem"] += 3
            waiters[s].extend(
                ag_send(chunk_lo, H1, units[0], g0, gbase0, sem_base, d)
                for d in (1, 2, 3)
            )
            ag_ctx[s] = (a2a, block2_lo, H1, g1, g0, gbase0, units[0])

        for d in (1, 2, 3):
            for s in (1, 0):
                a2a, block2_lo, H1, g1, g0, gbase0, u0 = ag_ctx[s]
                a2a[d - 1].wait()
                piece_lo = block2_lo + lax.rem(g1 + 4 - d, 4) * H1
                sem_base = ctr["sem"]
                ctr["sem"] += 3
                waiters[s].extend(
                    ag_send(piece_lo, H1, u0, g0, gbase0, sem_base, dd)
                    for dd in (1, 2, 3)
                )

        for s in (1, 0):
            for rdma in waiters[s]:
                rdma.wait()
            base, R, _ = STREAMS[s]
            out_ref[pl.ds(base, R), :] = ag_ref[
                pl.ds(base, R), :
            ].astype(jnp.float32)

    return pl.pallas_call(
        body,
        out_shape=jax.ShapeDtypeStruct((M, N), jnp.float32),
        in_specs=[
            pl.BlockSpec(memory_space=pltpu.VMEM),
            pl.BlockSpec(memory_space=pltpu.VMEM),
        ],
        out_specs=pl.BlockSpec(memory_space=pltpu.VMEM),
        scratch_shapes=[
            pltpu.VMEM((RS_ROWS, N), jnp.bfloat16),
            pltpu.VMEM((RS_ROWS, N), jnp.bfloat16),
            pltpu.VMEM((M, N), jnp.bfloat16),
            pltpu.SemaphoreType.DMA((N_EXCH,)),
            pltpu.SemaphoreType.DMA((N_EXCH,)),
        ],
        compiler_params=pltpu.CompilerParams(collective_id=0),
    )(t, W)


# device time: 26289 ns/iter; 1.1487x vs baseline; 1.0608x over previous
import jax
import jax.numpy as jnp
from jax import lax
from jax.experimental import pallas as pl
from jax.experimental.pallas import tpu as pltpu

N_DEV = 16
M = 1024
K = 512
N = 512

STREAMS = ((0, 640, (1, 4)), (640, 384, (4, 1)))
N_EXCH = 60
RS_ROWS = 3 * (160 + 40 + 96 + 24)


def kernel(t, W):
    def body(t_ref, w_ref, out_ref, stage_ref, comm_ref, ag_ref,
             send_sems, recv_sems):
        i = lax.axis_index("i")

        def group(u):
            g = lax.rem(lax.div(i, u), 4)
            return g, i - g * u

        def peer_at(u, g, gbase, d):
            return gbase + lax.rem(g + d, 4) * u

        barrier_sem = pltpu.get_barrier_semaphore()
        for u in (1, 4):
            g, gbase = group(u)
            for d in (1, 2, 3):
                pl.semaphore_signal(
                    barrier_sem, inc=1,
                    device_id=(peer_at(u, g, gbase, d),),
                    device_id_type=pl.DeviceIdType.MESH,
                )

        def mm(row_lo, rows):
            out_ref[pl.ds(row_lo, rows), :] = jnp.dot(
                t_ref[pl.ds(row_lo, rows), :], w_ref[...],
                preferred_element_type=jnp.float32,
            )

        ctr = {"sem": 0, "comm": 0, "stage": 0}

        stx = []
        for s, (base, R, units) in enumerate(STREAMS):
            H0, H1 = R // 4, R // 16
            g0, gbase0 = group(units[0])
            g1, gbase1 = group(units[1])
            keep_lo = base + g0 * H0
            c = {
                "base": base, "R": R, "units": units, "H0": H0,
                "H1": H1, "g0": g0, "gbase0": gbase0, "g1": g1,
                "gbase1": gbase1, "keep_lo": keep_lo,
                "sem0": ctr["sem"], "comm0": ctr["comm"],
                "stage0": ctr["stage"],
            }
            ctr["sem"] += 12
            ctr["comm"] += 3 * H0
            ctr["stage"] += 3 * H0
            c["sem1"] = ctr["sem"]
            ctr["sem"] += 3
            c["comm1"] = ctr["comm"]
            ctr["comm"] += 3 * H1
            c["stage1"] = ctr["stage"]
            ctr["stage"] += 3 * H1
            stx.append(c)

        for s, c in enumerate(stx):
            for d in (1, 2, 3):
                jm = lax.rem(c["g0"] + d, 4)
                q_lo = c["base"] + jm * c["H0"]
                mm(q_lo, c["H0"])
                stage_ref[
                    pl.ds(c["stage0"] + (d - 1) * c["H0"], c["H0"]), :
                ] = out_ref[pl.ds(q_lo, c["H0"]), :].astype(jnp.bfloat16)

        pl.semaphore_wait(barrier_sem, 6)

        rs0 = [[None] * 12 for _ in range(2)]
        for k in (1, 2, 3, 0):
            for s, c in enumerate(stx):
                j = lax.rem(c["g1"] + k, 4)
                off = j * c["H1"]
                for d in (1, 2, 3):
                    r = 4 - d
                    sem = c["sem0"] + (r - 1) * 4 + k
                    rdma = pltpu.make_async_remote_copy(
                        src_ref=stage_ref.at[
                            pl.ds(c["stage0"] + (d - 1) * c["H0"] + off,
                                  c["H1"]), :
                        ],
                        dst_ref=comm_ref.at[
                            pl.ds(c["comm0"] + (r - 1) * c["H0"] + off,
                                  c["H1"]), :
                        ],
                        send_sem=send_sems.at[sem],
                        recv_sem=recv_sems.at[sem],
                        device_id=(peer_at(c["units"][0], c["g0"],
                                           c["gbase0"], d),),
                        device_id_type=pl.DeviceIdType.MESH,
                    )
                    rdma.start()
                    rs0[s][(r - 1) * 4 + k] = rdma

        for s, c in enumerate(stx):
            mm(c["keep_lo"], c["H0"])

        rs1 = [[None] * 3 for _ in range(2)]
        for k in (1, 2, 3, 0):
            for s, c in enumerate(stx):
                H0, H1 = c["H0"], c["H1"]
                j = lax.rem(c["g1"] + k, 4)
                off = j * H1
                row = c["keep_lo"] + off
                for r in (1, 2, 3):
                    rs0[s][(r - 1) * 4 + k].wait()
                out_ref[pl.ds(row, H1), :] += (
                    comm_ref[pl.ds(c["comm0"] + off, H1), :].astype(
                        jnp.float32)
                    + comm_ref[pl.ds(c["comm0"] + H0 + off, H1), :].astype(
                        jnp.float32)
                    + comm_ref[
                        pl.ds(c["comm0"] + 2 * H0 + off, H1), :
                    ].astype(jnp.float32)
                )
                if k == 0:
                    continue
                r = 4 - k
                sem = c["sem1"] + r - 1
                st_off = c["stage1"] + (k - 1) * H1
                stage_ref[pl.ds(st_off, H1), :] = out_ref[
                    pl.ds(row, H1), :
                ].astype(jnp.bfloat16)
                rdma = pltpu.make_async_remote_copy(
                    src_ref=stage_ref.at[pl.ds(st_off, H1), :],
                    dst_ref=comm_ref.at[
                        pl.ds(c["comm1"] + (r - 1) * H1, H1), :
                    ],
                    send_sem=send_sems.at[sem],
                    recv_sem=recv_sems.at[sem],
                    device_id=(peer_at(c["units"][1], c["g1"],
                                       c["gbase1"], k),),
                    device_id_type=pl.DeviceIdType.MESH,
                )
                rdma.start()
                rs1[s][k - 1] = rdma

        def ag_send(c, src_lo, H, level, sem_base, d):
            u = c["units"][level]
            g = c["g0"] if level == 0 else c["g1"]
            gb = c["gbase0"] if level == 0 else c["gbase1"]
            r = 4 - d
            rdma = pltpu.make_async_remote_copy(
                src_ref=ag_ref.at[pl.ds(src_lo, H), :],
                dst_ref=ag_ref.at[pl.ds(src_lo, H), :],
                send_sem=send_sems.at[sem_base + r - 1],
                recv_sem=recv_sems.at[sem_base + r - 1],
                device_id=(peer_at(u, g, gb, d),),
                device_id_type=pl.DeviceIdType.MESH,
            )
            rdma.start()
            return rdma

        ag_ctx = [None, None]
        waiters = [[], []]
        for s, c in enumerate(stx):
            H1 = c["H1"]
            chunk_lo = c["keep_lo"] + c["g1"] * H1
            for rdma in rs1[s]:
                rdma.wait()
            out_ref[pl.ds(chunk_lo, H1), :] += (
                comm_ref[pl.ds(c["comm1"], H1), :].astype(jnp.float32)
                + comm_ref[pl.ds(c["comm1"] + H1, H1), :].astype(
                    jnp.float32)
                + comm_ref[pl.ds(c["comm1"] + 2 * H1, H1), :].astype(
                    jnp.float32)
            )
            ag_ref[pl.ds(chunk_lo, H1), :] = out_ref[
                pl.ds(chunk_lo, H1), :
            ].astype(jnp.bfloat16)
            sem_base = ctr["sem"]
            ctr["sem"] += 3
            a2a = [ag_send(c, chunk_lo, H1, 1, sem_base, d)
                   for d in (1, 2, 3)]
            sem_base = ctr["sem"]
            ctr["sem"] += 3
            waiters[s].extend(
                ag_send(c, chunk_lo, H1, 0, sem_base, d)
                for d in (1, 2, 3)
            )
            ag_ctx[s] = (a2a, c["keep_lo"])

        for d in (1, 2, 3):
            for s in (1, 0):
                c = stx[s]
                a2a, block2_lo = ag_ctx[s]
                a2a[d - 1].wait()
                piece_lo = block2_lo + lax.rem(c["g1"] + 4 - d, 4) * c["H1"]
                sem_base = ctr["sem"]
                ctr["sem"] += 3
                waiters[s].extend(
                    ag_send(c, piece_lo, c["H1"], 0, sem_base, dd)
                    for dd in (1, 2, 3)
                )

        for s in (1, 0):
            for rdma in waiters[s]:
                rdma.wait()
            base, R, _ = STREAMS[s]
            out_ref[pl.ds(base, R), :] = ag_ref[
                pl.ds(base, R), :
            ].astype(jnp.float32)

    return pl.pallas_call(
        body,
        out_shape=jax.ShapeDtypeStruct((M, N), jnp.float32),
        in_specs=[
            pl.BlockSpec(memory_space=pltpu.VMEM),
            pl.BlockSpec(memory_space=pltpu.VMEM),
        ],
        out_specs=pl.BlockSpec(memory_space=pltpu.VMEM),
        scratch_shapes=[
            pltpu.VMEM((RS_ROWS, N), jnp.bfloat16),
            pltpu.VMEM((RS_ROWS, N), jnp.bfloat16),
            pltpu.VMEM((M, N), jnp.bfloat16),
            pltpu.SemaphoreType.DMA((N_EXCH,)),
            pltpu.SemaphoreType.DMA((N_EXCH,)),
        ],
        compiler_params=pltpu.CompilerParams(collective_id=0),
    )(t, W)
